# baseline (speedup 1.0000x reference)
import functools

import jax
import jax.numpy as jnp
from jax import lax
from jax.experimental import pallas as pl
from jax.experimental.pallas import tpu as pltpu
from jax.experimental.pallas import tpu_sc as plsc

EPS_LN = 1e-12

_NC = 2
_NS = 16
_NW = _NC * _NS
_GW = 16
_TB = 256


def _sc_gather(word_emb, flat_ids):
    n = flat_ids.shape[0]
    _, d = word_emb.shape
    b_per_w = n // _NW
    nchunks = b_per_w // _GW
    mesh = plsc.VectorSubcoreMesh(core_axis_name="c", subcore_axis_name="s")

    @functools.partial(
        pl.kernel,
        mesh=mesh,
        out_type=jax.ShapeDtypeStruct((n, d), word_emb.dtype),
        scratch_types=[
            pltpu.VMEM((b_per_w,), jnp.int32),
            pltpu.VMEM((_GW, d), word_emb.dtype),
            pltpu.SemaphoreType.DMA,
        ],
    )
    def gather_kernel(table_hbm, idx_hbm, out_hbm, idx_v, rows_v, sem):
        wid = lax.axis_index("s") * _NC + lax.axis_index("c")
        base = wid * b_per_w
        pltpu.sync_copy(idx_hbm.at[pl.ds(base, b_per_w)], idx_v)

        @pl.loop(0, nchunks)
        def _(c):
            off = c * _GW
            pltpu.async_copy(
                table_hbm.at[idx_v.at[pl.ds(off, _GW)]], rows_v, sem
            ).wait()
            pltpu.sync_copy(rows_v, out_hbm.at[pl.ds(base + off, _GW)])

    return gather_kernel(word_emb, flat_ids)


def _ln_body_first(g_ref, pos_ref, tt_ref, tok_ref, gam_ref, bet_ref, o_ref):
    e = g_ref[...] + pos_ref[...]
    w = tt_ref[...]
    e = e + (tok_ref[0:1, :] + w * (tok_ref[1:2, :] - tok_ref[0:1, :]))
    mu = jnp.mean(e, axis=1, keepdims=True)
    m2 = jnp.mean(e * e, axis=1, keepdims=True)
    a = lax.rsqrt(m2 - mu * mu + EPS_LN)
    o_ref[...] = (e - mu) * a * gam_ref[...] + bet_ref[...]


def _ln_body_next(prev_ref, g_ref, pos_ref, tt_ref, tok_ref, gam_ref, bet_ref,
                  o_ref):
    del prev_ref
    _ln_body_first(g_ref, pos_ref, tt_ref, tok_ref, gam_ref, bet_ref, o_ref)


def _tc_ln_row(out_prev, gathered, tt_w, pos_emb, tok_emb, gamma2d, beta2d,
               row, total_n):
    s, h = gathered.shape
    n_s = s // _TB
    grid = (n_s,)
    data_specs = [
        pl.BlockSpec((_TB, h), lambda i: (i, 0)),
        pl.BlockSpec((_TB, h), lambda i: (i, 0)),
        pl.BlockSpec((_TB, 1), lambda i: (i, 0)),
        pl.BlockSpec(tok_emb.shape, lambda i: (0, 0)),
        pl.BlockSpec((1, h), lambda i: (0, 0)),
        pl.BlockSpec((1, h), lambda i: (0, 0)),
    ]
    out_spec = pl.BlockSpec((_TB, h), lambda i, r=row, ns=n_s: (r * ns + i, 0))
    out_shape = jax.ShapeDtypeStruct((total_n, h), jnp.float32)
    data = (gathered, pos_emb, tt_w, tok_emb, gamma2d, beta2d)
    if out_prev is None:
        return pl.pallas_call(
            _ln_body_first,
            grid=grid,
            in_specs=data_specs,
            out_specs=out_spec,
            out_shape=out_shape,
        )(*data)
    return pl.pallas_call(
        _ln_body_next,
        grid=grid,
        in_specs=[pl.BlockSpec(memory_space=pl.ANY)] + data_specs,
        out_specs=out_spec,
        out_shape=out_shape,
        input_output_aliases={0: 0},
    )(out_prev, *data)


def kernel(input_ids, token_type_ids, word_emb, pos_emb, tok_emb, gamma, beta):
    b, s = input_ids.shape
    h = word_emb.shape[1]
    ids = input_ids.astype(jnp.int32)
    tt_w = token_type_ids.astype(jnp.float32).reshape(b, s, 1)
    pos = pos_emb[:s]
    gamma2d = gamma.reshape(1, -1)
    beta2d = beta.reshape(1, -1)
    out = None
    for row in range(b):
        gathered = _sc_gather(word_emb, ids[row])
        out = _tc_ln_row(out, gathered, tt_w[row], pos, tok_emb, gamma2d,
                         beta2d, row, b * s)
    return out.reshape(b, s, h)


# speedup vs baseline: 1.0986x; 1.0986x over previous
import functools

import jax
import jax.numpy as jnp
from jax import lax
from jax.experimental import pallas as pl
from jax.experimental.pallas import tpu as pltpu
from jax.experimental.pallas import tpu_sc as plsc

EPS_LN = 1e-12

_NC = 2
_NS = 16
_NW = _NC * _NS
_GW = 16
_TB = 256
_NCH = 4


def _sc_gather(word_emb, flat_ids):
    n = flat_ids.shape[0]
    _, d = word_emb.shape
    b_per_w = n // _NW
    nsub = b_per_w // _GW
    mesh = plsc.VectorSubcoreMesh(core_axis_name="c", subcore_axis_name="s")

    @functools.partial(
        pl.kernel,
        mesh=mesh,
        out_type=jax.ShapeDtypeStruct((n, d), word_emb.dtype),
        scratch_types=[
            pltpu.VMEM((b_per_w,), jnp.int32),
            pltpu.VMEM((_GW, d), word_emb.dtype),
            pltpu.VMEM((_GW, d), word_emb.dtype),
            pltpu.SemaphoreType.DMA,
            pltpu.SemaphoreType.DMA,
            pltpu.SemaphoreType.DMA,
            pltpu.SemaphoreType.DMA,
        ],
    )
    def gather_kernel(table_hbm, idx_hbm, out_hbm, idx_v, r0, r1, gs0, gs1,
                      os0, os1):
        wid = lax.axis_index("s") * _NC + lax.axis_index("c")
        base = wid * b_per_w
        pltpu.sync_copy(idx_hbm.at[pl.ds(base, b_per_w)], idx_v)
        bufs = (r0, r1)
        gsems = (gs0, gs1)
        osems = (os0, os1)

        def gath(j):
            return pltpu.make_async_copy(
                table_hbm.at[idx_v.at[pl.ds(j * _GW, _GW)]],
                bufs[j % 2],
                gsems[j % 2],
            )

        def wr(j):
            return pltpu.make_async_copy(
                bufs[j % 2],
                out_hbm.at[pl.ds(base + j * _GW, _GW)],
                osems[j % 2],
            )

        gath(0).start()
        if nsub > 1:
            gath(1).start()
        for j in range(nsub):
            gath(j).wait()
            wr(j).start()
            if j >= 1 and j + 1 < nsub:
                wr(j - 1).wait()
                gath(j + 1).start()
        if nsub > 1:
            wr(nsub - 2).wait()
        wr(nsub - 1).wait()

    return gather_kernel(word_emb, flat_ids)


def _ln_body_first(g_ref, pos_ref, tt_ref, tok_ref, gam_ref, bet_ref, o_ref):
    e = g_ref[...] + pos_ref[...]
    w = tt_ref[...]
    e = e + (tok_ref[0:1, :] + w * (tok_ref[1:2, :] - tok_ref[0:1, :]))
    mu = jnp.mean(e, axis=1, keepdims=True)
    m2 = jnp.mean(e * e, axis=1, keepdims=True)
    a = lax.rsqrt(m2 - mu * mu + EPS_LN)
    o_ref[...] = (e - mu) * a * gam_ref[...] + bet_ref[...]


def _ln_body_next(prev_ref, g_ref, pos_ref, tt_ref, tok_ref, gam_ref, bet_ref,
                  o_ref):
    del prev_ref
    _ln_body_first(g_ref, pos_ref, tt_ref, tok_ref, gam_ref, bet_ref, o_ref)


def _tc_ln_chunk(out_prev, gathered, tt_w, pos_emb, tok_emb, gamma2d, beta2d,
                 chunk, n_s_total, batch, total_n):
    n_ck, h = gathered.shape
    sw = n_ck // batch
    n_sc = sw // _TB
    grid = (n_sc, batch)
    data_specs = [
        pl.BlockSpec((_TB, h), lambda i, bb: (bb * n_sc + i, 0)),
        pl.BlockSpec((_TB, h), lambda i, bb: (chunk * n_sc + i, 0)),
        pl.BlockSpec((_TB, 1), lambda i, bb: (bb * n_sc + i, 0)),
        pl.BlockSpec(tok_emb.shape, lambda i, bb: (0, 0)),
        pl.BlockSpec((1, h), lambda i, bb: (0, 0)),
        pl.BlockSpec((1, h), lambda i, bb: (0, 0)),
    ]
    out_spec = pl.BlockSpec(
        (_TB, h), lambda i, bb: (bb * n_s_total + chunk * n_sc + i, 0)
    )
    out_shape = jax.ShapeDtypeStruct((total_n, h), jnp.float32)
    data = (gathered, pos_emb, tt_w, tok_emb, gamma2d, beta2d)
    if out_prev is None:
        return pl.pallas_call(
            _ln_body_first,
            grid=grid,
            in_specs=data_specs,
            out_specs=out_spec,
            out_shape=out_shape,
        )(*data)
    return pl.pallas_call(
        _ln_body_next,
        grid=grid,
        in_specs=[pl.BlockSpec(memory_space=pl.ANY)] + data_specs,
        out_specs=out_spec,
        out_shape=out_shape,
        input_output_aliases={0: 0},
    )(out_prev, *data)


def kernel(input_ids, token_type_ids, word_emb, pos_emb, tok_emb, gamma, beta):
    b, s = input_ids.shape
    h = word_emb.shape[1]
    sw = s // _NCH
    ids = input_ids.astype(jnp.int32)
    tt_w = token_type_ids.astype(jnp.float32)
    pos = pos_emb[:s]
    gamma2d = gamma.reshape(1, -1)
    beta2d = beta.reshape(1, -1)
    n_s_total = s // _TB
    out = None
    for c in range(_NCH):
        ids_c = ids[:, c * sw:(c + 1) * sw].reshape(-1)
        tt_c = tt_w[:, c * sw:(c + 1) * sw].reshape(-1, 1)
        gathered = _sc_gather(word_emb, ids_c)
        out = _tc_ln_chunk(out, gathered, tt_c, pos, tok_emb, gamma2d, beta2d,
                           c, n_s_total, b, b * s)
    return out.reshape(b, s, h)
